# baseline (speedup 1.0000x reference)
import functools

import jax
import jax.numpy as jnp
from jax import lax
from jax.experimental import pallas as pl
from jax.experimental.pallas import tpu as pltpu
from jax.experimental.pallas import tpu_sc as plsc

S = 2048
CLIP = 64
D = 64
T = 2 * CLIP + 1
NTD = D // 8
NTJ = S // 128
KW = 400
SRC_MAX = 272
BAND_LO = 128
BAND_HI = 272
NW = 32
ROWS_PER_W = S // NW
NPH = 8
RPG = ROWS_PER_W // NPH
BPR = 2
CREP = 4


def _build_sc_kernel():
    mesh = plsc.VectorSubcoreMesh(core_axis_name="c", subcore_axis_name="s")

    @functools.partial(
        pl.kernel,
        mesh=mesh,
        out_type=jax.ShapeDtypeStruct((S, NTD, NTJ, 8, 128), jnp.float32),
        scratch_types=[
            pltpu.VMEM((T * D,), jnp.float32),
            pltpu.VMEM((NTD, 8, KW), jnp.float32),
            pltpu.VMEM((NTD, 8, KW), jnp.float32),
            pltpu.VMEM((NTD, CREP, 8, 128), jnp.float32),
            pltpu.VMEM((NTD, CREP, 8, 128), jnp.float32),
            pltpu.SemaphoreType.DMA,
            pltpu.SemaphoreType.DMA,
        ],
        compiler_params=pltpu.CompilerParams(
            use_tc_tiling_on_sc=False, needs_layout_passes=False
        ),
    )
    def sc_kernel(table_hbm, out_hbm, tbl, cimg_a, cimg_b, c0, c128, semb, semc):
        cid = lax.axis_index("c")
        sid = lax.axis_index("s")
        wid = sid * 2 + cid
        bufs = (cimg_a, cimg_b)
        i0 = wid * ROWS_PER_W

        pltpu.sync_copy(table_hbm, tbl)

        lane = lax.iota(jnp.int32, 16)

        def const_fill(q, _):
            dcomb = q // (CREP * 8)
            r = q % (CREP * 8)
            tjj = r // 8
            lb = r % 8
            v0 = plsc.load_gather(tbl, [jnp.full((16,), dcomb, jnp.int32)])
            v1 = plsc.load_gather(
                tbl, [jnp.full((16,), (T - 1) * D + dcomb, jnp.int32)]
            )
            c0[dcomb // 8, tjj, dcomb % 8, pl.ds(lb * 16, 16)] = v0
            c128[dcomb // 8, tjj, dcomb % 8, pl.ds(lb * 16, 16)] = v1
            return 0

        lax.fori_loop(0, D * CREP * 8, const_fill, 0)

        def build(buf, p, m_lo, m_hi):
            ng = (m_hi - m_lo) // 16

            def body(g, _):
                dcomb = g // ng
                mg = g % ng
                m = m_lo + mg * 16 + lane
                row = jnp.clip(m + p - 136, 0, T - 1)
                vals = plsc.load_gather(tbl, [row * D + dcomb])
                buf[dcomb // 8, dcomb % 8, pl.ds(m_lo + mg * 16, 16)] = vals
                return 0

            lax.fori_loop(0, D * ng, body, 0)

        build(bufs[0], 0, 0, KW)
        build(bufs[1], 7, 0, KW)

        def const_body(r, cnt):
            i = i0 + r
            t_lo = jnp.maximum(0, (i - 63) // 128)
            t_hi = jnp.minimum(NTJ, (i + 191) // 128)
            n_l = t_lo
            n_r = NTJ - t_hi

            def left4(q, _):
                pltpu.async_copy(
                    c0.at[:, pl.ds(0, 4)], out_hbm.at[i, :, pl.ds(q * 4, 4)], semc
                )
                return 0

            lax.fori_loop(0, n_l // 4, left4, 0)

            @pl.when((n_l & 2) != 0)
            def _():
                pltpu.async_copy(
                    c0.at[:, pl.ds(0, 2)],
                    out_hbm.at[i, :, pl.ds((n_l // 4) * 4, 2)],
                    semc,
                )

            @pl.when((n_l & 1) != 0)
            def _():
                pltpu.async_copy(
                    c0.at[:, pl.ds(0, 1)],
                    out_hbm.at[i, :, pl.ds(n_l - 1, 1)],
                    semc,
                )

            def right4(q, _):
                pltpu.async_copy(
                    c128.at[:, pl.ds(0, 4)],
                    out_hbm.at[i, :, pl.ds(t_hi + q * 4, 4)],
                    semc,
                )
                return 0

            lax.fori_loop(0, n_r // 4, right4, 0)

            @pl.when((n_r & 2) != 0)
            def _():
                pltpu.async_copy(
                    c128.at[:, pl.ds(0, 2)],
                    out_hbm.at[i, :, pl.ds(t_hi + (n_r // 4) * 4, 2)],
                    semc,
                )

            @pl.when((n_r & 1) != 0)
            def _():
                pltpu.async_copy(
                    c128.at[:, pl.ds(0, 1)],
                    out_hbm.at[i, :, pl.ds(NTJ - 1, 1)],
                    semc,
                )

            return cnt + n_l + n_r

        cunits = lax.fori_loop(0, ROWS_PER_W, const_body, 0)

        def drain_band_group():
            def body(q, _):
                pltpu.make_async_copy(
                    bufs[0].at[:, :, pl.ds(0, 128)],
                    out_hbm.at[i0, :, 0],
                    semb,
                ).wait()
                return 0

            lax.fori_loop(0, RPG * BPR, body, 0)

        for g in range(NPH):
            p = (-g) % NPH
            buf = bufs[g % 2]

            def band_body(q, _, g=g, p=p, buf=buf):
                rr = q // BPR
                k = q % BPR
                i = i0 + rr * NPH + g
                t_lo = jnp.maximum(0, (i - 63) // 128)
                t_hi = jnp.minimum(NTJ, (i + 191) // 128)
                tj = jnp.where(k == 0, t_lo, t_hi - 1)
                src = jnp.clip(S - i + tj * 128 - 1848 - p, 0, SRC_MAX)
                src = pl.multiple_of(src, 8)
                pltpu.async_copy(
                    buf.at[:, :, pl.ds(src, 128)],
                    out_hbm.at[i, :, tj],
                    semb,
                )
                return 0

            lax.fori_loop(0, RPG * BPR, band_body, 0)

            if g >= 1:
                drain_band_group()
            if 1 <= g < NPH - 1:
                build(bufs[(g + 1) % 2], (-(g + 1)) % NPH, BAND_LO, BAND_HI)

        drain_band_group()

        def const_drain(q, _):
            pltpu.make_async_copy(
                c0.at[:, pl.ds(0, 1)],
                out_hbm.at[i0, :, pl.ds(0, 1)],
                semc,
            ).wait()
            return 0

        lax.fori_loop(0, cunits, const_drain, 0)

    return sc_kernel


def kernel(x, encoding_matrix):
    del x
    b = _build_sc_kernel()(encoding_matrix.reshape(T * D))
    return b.transpose(0, 2, 4, 1, 3).reshape(S, S, D)


# speedup vs baseline: 1.0612x; 1.0612x over previous
import functools

import jax
import jax.numpy as jnp
from jax import lax
from jax.experimental import pallas as pl
from jax.experimental.pallas import tpu as pltpu
from jax.experimental.pallas import tpu_sc as plsc

S = 2048
CLIP = 64
D = 64
T = 2 * CLIP + 1
NTD = D // 8
NTJ = S // 128
KW = 400
SRC_MAX = 272
BAND_LO = 128
BAND_HI = 272
NW = 32
ROWS_PER_W = S // NW
NPH = 8
RPG = ROWS_PER_W // NPH


def _build_sc_kernel():
    mesh = plsc.VectorSubcoreMesh(core_axis_name="c", subcore_axis_name="s")

    @functools.partial(
        pl.kernel,
        mesh=mesh,
        out_type=jax.ShapeDtypeStruct((S, NTD, NTJ, 8, 128), jnp.float32),
        scratch_types=[
            pltpu.VMEM((T * D,), jnp.float32),
            pltpu.VMEM((NTD, 8, KW), jnp.float32),
            pltpu.VMEM((NTD, 8, KW), jnp.float32),
            pltpu.SemaphoreType.DMA,
        ],
        compiler_params=pltpu.CompilerParams(
            use_tc_tiling_on_sc=False, needs_layout_passes=False
        ),
    )
    def sc_kernel(table_hbm, out_hbm, tbl, cimg_a, cimg_b, sem):
        cid = lax.axis_index("c")
        sid = lax.axis_index("s")
        wid = sid * 2 + cid
        bufs = (cimg_a, cimg_b)

        pltpu.sync_copy(table_hbm, tbl)

        lane = lax.iota(jnp.int32, 16)

        def build(buf, p, m_lo, m_hi):
            ng = (m_hi - m_lo) // 16

            def body(g, _):
                dcomb = g // ng
                mg = g % ng
                m = m_lo + mg * 16 + lane
                row = jnp.clip(m + p - 136, 0, T - 1)
                vals = plsc.load_gather(tbl, [row * D + dcomb])
                buf[dcomb // 8, dcomb % 8, pl.ds(m_lo + mg * 16, 16)] = vals
                return 0

            lax.fori_loop(0, D * ng, body, 0)

        def drain_group():
            def body(q, _):
                pltpu.make_async_copy(
                    bufs[0].at[:, :, pl.ds(0, 128)],
                    out_hbm.at[i0, :, 0],
                    sem,
                ).wait()
                return 0

            lax.fori_loop(0, RPG * NTJ, body, 0)

        i0 = wid * ROWS_PER_W

        build(bufs[0], 0, 0, KW)
        build(bufs[1], 7, 0, KW)

        for g in range(NPH):
            p = (-g) % NPH
            buf = bufs[g % 2]
            if g >= 1:
                drain_group()

            def issue_body(q, _, g=g, p=p, buf=buf):
                rr = q // NTJ
                tj = q % NTJ
                i = i0 + rr * NPH + g
                src = jnp.clip(S - i + tj * 128 - 1848 - p, 0, SRC_MAX)
                src = pl.multiple_of(src, 8)
                pltpu.async_copy(
                    buf.at[:, :, pl.ds(src, 128)],
                    out_hbm.at[i, :, tj],
                    sem,
                )
                return 0

            lax.fori_loop(0, RPG * NTJ, issue_body, 0)

            if 1 <= g < NPH - 1:
                build(bufs[(g + 1) % 2], (-(g + 1)) % NPH, BAND_LO, BAND_HI)

        drain_group()

    return sc_kernel


def kernel(x, encoding_matrix):
    del x
    b = _build_sc_kernel()(encoding_matrix.reshape(T * D))
    return b.transpose(0, 2, 4, 1, 3).reshape(S, S, D)
